# baseline (speedup 1.0000x reference)
import functools

import jax
import jax.numpy as jnp
from jax import lax
from jax.experimental import pallas as pl
from jax.experimental.pallas import tpu as pltpu
from jax.experimental.pallas import tpu_sc as plsc

B = 16384
VOCAB = 1000000
DIM = 64
NEG = 512
CHUNKS = B // NEG
GCH = 128
CONV_CB = 32768


@functools.lru_cache(maxsize=None)
def _make_sc_gather(nw: int, nb: int):
    b_per_w = nb // nw
    n_gch = b_per_w // GCH
    mesh = plsc.VectorSubcoreMesh(core_axis_name="c", subcore_axis_name="s")
    nc = plsc.get_sparse_core_info().num_cores

    @functools.partial(
        pl.kernel,
        mesh=mesh,
        out_type=[
            jax.ShapeDtypeStruct((nb, 2 * DIM), jnp.float32),
            jax.ShapeDtypeStruct((nb, 2 * DIM), jnp.float32),
        ],
        scratch_types=[
            pltpu.VMEM((b_per_w,), jnp.int32),
            pltpu.VMEM((b_per_w,), jnp.int32),
            pltpu.VMEM((2 * GCH, 2 * DIM), jnp.float32),
            pltpu.VMEM((2 * GCH, 2 * DIM), jnp.float32),
            pltpu.SemaphoreType.DMA,
            pltpu.SemaphoreType.DMA,
        ],
    )
    def gather_kernel(lidx_hbm, ridx_hbm, t2_hbm, lhs_out, rhs_out,
                      lidx_v, ridx_v, lbuf, rbuf, gsem, wsem):
        wid = lax.axis_index("s") * nc + lax.axis_index("c")
        base = wid * b_per_w
        pltpu.sync_copy(lidx_hbm.at[wid], lidx_v)
        pltpu.sync_copy(ridx_hbm.at[wid], ridx_v)

        units = []
        for g in range(n_gch):
            units.append((lidx_v, lbuf, lhs_out, g))
            units.append((ridx_v, rbuf, rhs_out, g))

        def fire(u):
            idx_v, buf, out_hbm, g = u
            slot = (g % 2) * GCH
            return pltpu.async_copy(
                t2_hbm.at[idx_v.at[pl.ds(g * GCH, GCH)]],
                buf.at[pl.ds(slot, GCH)], gsem)

        gdescs = [fire(units[0]), fire(units[1]), fire(units[2]),
                  fire(units[3])]
        wdescs = []
        for u in range(len(units)):
            idx_v, buf, out_hbm, g = units[u]
            slot = (g % 2) * GCH
            gdescs[u].wait()
            wdescs.append(pltpu.async_copy(
                buf.at[pl.ds(slot, GCH)],
                out_hbm.at[pl.ds(base + g * GCH, GCH)], wsem))
            if u + 4 < len(units):
                wdescs[u].wait()
                gdescs.append(fire(units[u + 4]))
        for d in wdescs[-4:]:
            d.wait()

    return gather_kernel


def _tc_convert(tt, eye2):
    grid = (VOCAB + CONV_CB - 1) // CONV_CB

    def body(tt_ref, eye_ref, out_ref):
        out_ref[...] = lax.dot_general(
            tt_ref[...], eye_ref[...],
            (((0,), (0,)), ((), ())),
            preferred_element_type=jnp.float32,
        )

    return pl.pallas_call(
        body,
        grid=(grid,),
        compiler_params=pltpu.CompilerParams(
            vmem_limit_bytes=128 * 1024 * 1024),
        in_specs=[
            pl.BlockSpec((DIM, CONV_CB), lambda c: (0, c)),
            pl.BlockSpec((DIM, 2 * DIM), lambda c: (0, 0)),
        ],
        out_specs=pl.BlockSpec((CONV_CB, 2 * DIM), lambda c: (c, 0)),
        out_shape=jax.ShapeDtypeStruct((VOCAB, 2 * DIM), jnp.float32),
    )(tt, eye2)


def _tc_scores(lhs_rows, rhs_rows, half, prev=None):
    def body(*refs):
        lp_ref, rp_ref, out_ref = refs[0], refs[1], refs[-1]
        for h in range(8):
            sl = pl.ds(h * NEG, NEG)
            out_ref[h] = lax.dot_general(
                lp_ref[sl, DIM:], rp_ref[sl, :DIM],
                (((1,), (1,)), ((), ())),
                preferred_element_type=jnp.float32,
            )

    nsteps = CHUNKS // 16
    off = half * nsteps
    in_specs = [
        pl.BlockSpec((8 * NEG, 2 * DIM), lambda c: (c, 0)),
        pl.BlockSpec((8 * NEG, 2 * DIM), lambda c: (c, 0)),
    ]
    args = [lhs_rows, rhs_rows]
    kwargs = {}
    if prev is not None:
        in_specs.append(pl.BlockSpec(memory_space=pl.ANY))
        args.append(prev)
        kwargs["input_output_aliases"] = {2: 0}
    return pl.pallas_call(
        body,
        grid=(nsteps,),
        in_specs=in_specs,
        out_specs=pl.BlockSpec((8, NEG, NEG), lambda c: (c + off, 0, 0)),
        out_shape=jax.ShapeDtypeStruct((CHUNKS, NEG, NEG), jnp.float32),
        **kwargs,
    )(*args)


def kernel(lhs_idx, rhs_idx, emb_table, rel_vec):
    info = plsc.get_sparse_core_info()
    nw = info.num_cores * info.num_subcores
    b_per_w = B // nw
    eye = jnp.eye(DIM, dtype=jnp.float32)
    eye2 = jnp.concatenate([eye, eye * rel_vec[None, :]], axis=1)
    t2 = _tc_convert(emb_table.T, eye2)
    lidx = lhs_idx.astype(jnp.int32)
    ridx = rhs_idx.astype(jnp.int32)
    nb = B // 2
    gather = _make_sc_gather(nw, nb)
    l2 = lidx.reshape(2, nw, nb // nw)
    r2 = ridx.reshape(2, nw, nb // nw)
    lhsA, rhsA = gather(l2[0], r2[0], t2)
    lhsB, rhsB = gather(l2[1], r2[1], t2)
    scoresA = _tc_scores(lhsA, rhsA, 0)
    return _tc_scores(lhsB, rhsB, 1, prev=scoresA)


# speedup vs baseline: 1.0111x; 1.0111x over previous
import functools

import jax
import jax.numpy as jnp
from jax import lax
from jax.experimental import pallas as pl
from jax.experimental.pallas import tpu as pltpu
from jax.experimental.pallas import tpu_sc as plsc

B = 16384
VOCAB = 1000000
DIM = 64
NEG = 512
CHUNKS = B // NEG
GCH = 128
CONV_CB = 32768


@functools.lru_cache(maxsize=None)
def _make_sc_gather(nw: int):
    b_per_w = B // nw
    n_gch = b_per_w // GCH
    mesh = plsc.VectorSubcoreMesh(core_axis_name="c", subcore_axis_name="s")
    nc = plsc.get_sparse_core_info().num_cores

    @functools.partial(
        pl.kernel,
        mesh=mesh,
        out_type=[
            jax.ShapeDtypeStruct((B, 2 * DIM), jnp.float32),
            jax.ShapeDtypeStruct((B, 2 * DIM), jnp.float32),
        ],
        scratch_types=[
            pltpu.VMEM((b_per_w,), jnp.int32),
            pltpu.VMEM((b_per_w,), jnp.int32),
            pltpu.VMEM((2 * GCH, 2 * DIM), jnp.float32),
            pltpu.VMEM((2 * GCH, 2 * DIM), jnp.float32),
            pltpu.SemaphoreType.DMA,
            pltpu.SemaphoreType.DMA,
        ],
    )
    def gather_kernel(lidx_hbm, ridx_hbm, t2_hbm, lhs_out, rhs_out,
                      lidx_v, ridx_v, lbuf, rbuf, gsem, wsem):
        wid = lax.axis_index("s") * nc + lax.axis_index("c")
        base = wid * b_per_w
        pltpu.sync_copy(lidx_hbm.at[wid], lidx_v)
        pltpu.sync_copy(ridx_hbm.at[wid], ridx_v)

        units = []
        for g in range(n_gch):
            units.append((lidx_v, lbuf, lhs_out, g))
            units.append((ridx_v, rbuf, rhs_out, g))

        def fire(u):
            idx_v, buf, out_hbm, g = u
            slot = (g % 2) * GCH
            return pltpu.async_copy(
                t2_hbm.at[idx_v.at[pl.ds(g * GCH, GCH)]],
                buf.at[pl.ds(slot, GCH)], gsem)

        gdescs = [fire(units[0]), fire(units[1]), fire(units[2]),
                  fire(units[3])]
        wdescs = []
        for u in range(len(units)):
            idx_v, buf, out_hbm, g = units[u]
            slot = (g % 2) * GCH
            gdescs[u].wait()
            wdescs.append(pltpu.async_copy(
                buf.at[pl.ds(slot, GCH)],
                out_hbm.at[pl.ds(base + g * GCH, GCH)], wsem))
            if u + 4 < len(units):
                wdescs[u].wait()
                gdescs.append(fire(units[u + 4]))
        for d in wdescs[-4:]:
            d.wait()

    return gather_kernel


def _tc_convert(tt, eye2):
    grid = (VOCAB + CONV_CB - 1) // CONV_CB

    def body(tt_ref, eye_ref, out_ref):
        out_ref[...] = lax.dot_general(
            tt_ref[...], eye_ref[...],
            (((0,), (0,)), ((), ())),
            preferred_element_type=jnp.float32,
        )

    return pl.pallas_call(
        body,
        grid=(grid,),
        compiler_params=pltpu.CompilerParams(
            vmem_limit_bytes=128 * 1024 * 1024),
        in_specs=[
            pl.BlockSpec((DIM, CONV_CB), lambda c: (0, c)),
            pl.BlockSpec((DIM, 2 * DIM), lambda c: (0, 0)),
        ],
        out_specs=pl.BlockSpec((CONV_CB, 2 * DIM), lambda c: (c, 0)),
        out_shape=jax.ShapeDtypeStruct((VOCAB, 2 * DIM), jnp.float32),
    )(tt, eye2)


def _tc_scores(lhs_rows, rhs_rows):
    def body(lp_ref, rp_ref, out_ref):
        for h in range(8):
            sl = pl.ds(h * NEG, NEG)
            out_ref[h] = lax.dot_general(
                lp_ref[sl, DIM:], rp_ref[sl, :DIM],
                (((1,), (1,)), ((), ())),
                preferred_element_type=jnp.float32,
            )

    return pl.pallas_call(
        body,
        grid=(CHUNKS // 8,),
        in_specs=[
            pl.BlockSpec((8 * NEG, 2 * DIM), lambda c: (c, 0)),
            pl.BlockSpec((8 * NEG, 2 * DIM), lambda c: (c, 0)),
        ],
        out_specs=pl.BlockSpec((8, NEG, NEG), lambda c: (c, 0, 0)),
        out_shape=jax.ShapeDtypeStruct((CHUNKS, NEG, NEG), jnp.float32),
    )(lhs_rows, rhs_rows)


def kernel(lhs_idx, rhs_idx, emb_table, rel_vec):
    info = plsc.get_sparse_core_info()
    nw = info.num_cores * info.num_subcores
    b_per_w = B // nw
    eye = jnp.eye(DIM, dtype=jnp.float32)
    eye2 = jnp.concatenate([eye, eye * rel_vec[None, :]], axis=1)
    t2 = _tc_convert(emb_table.T, eye2)
    lidx = lhs_idx.astype(jnp.int32)
    ridx = rhs_idx.astype(jnp.int32)
    lhs_rows, rhs_rows = _make_sc_gather(nw)(
        lidx.reshape(nw, b_per_w), ridx.reshape(nw, b_per_w), t2)
    return _tc_scores(lhs_rows, rhs_rows)
